# baseline (speedup 1.0000x reference)
import jax
import jax.numpy as jnp
from jax.experimental import pallas as pl
from jax.experimental.pallas import tpu as pltpu

_P = 256
_QB = 1920
_BIG = 1e9


def _chamfer_body(bc_ref, t_ref, out_ref):
    bc = bc_ref[0]
    q = t_ref.shape[2]
    nchunks = q // _QB

    def body(j, carry):
        run_min, acc_y, acc_len = carry
        tj = t_ref[0, :, pl.ds(j * _QB, _QB)]
        mask = tj >= 0.001
        tx = jnp.where(mask, tj, _BIG)
        d = (bc - tx) ** 2
        run_min = jnp.minimum(run_min, d)
        dy = jnp.min(d, axis=0, keepdims=True)
        acc_y = acc_y + jnp.where(mask, dy, 0.0)
        acc_len = acc_len + mask.astype(jnp.float32)
        return run_min, acc_y, acc_len

    init = (
        jnp.full((_P, _QB), jnp.inf, jnp.float32),
        jnp.zeros((1, _QB), jnp.float32),
        jnp.zeros((1, _QB), jnp.float32),
    )
    run_min, acc_y, acc_len = jax.lax.fori_loop(0, nchunks, body, init)
    cham_x = jnp.sum(jnp.min(run_min, axis=1)) / _P
    cham_y = jnp.sum(acc_y) / jnp.maximum(jnp.sum(acc_len), 1.0)
    out_ref[0] = jnp.full((1, 128), cham_x + cham_y, jnp.float32)


def kernel(bins, target_depth_maps):
    n = bins.shape[0]
    q = target_depth_maps.shape[1] * target_depth_maps.shape[2]
    bc = 0.5 * (bins[:, 1:] + bins[:, :-1])
    bc3 = bc.reshape(n, _P, 1)
    t3 = target_depth_maps.reshape(n, 1, q)
    per_batch = pl.pallas_call(
        _chamfer_body,
        grid=(n,),
        in_specs=[
            pl.BlockSpec((1, _P, 1), lambda i: (i, 0, 0)),
            pl.BlockSpec((1, 1, q), lambda i: (i, 0, 0)),
        ],
        out_specs=pl.BlockSpec((1, 1, 128), lambda i: (i, 0, 0)),
        out_shape=jax.ShapeDtypeStruct((n, 1, 128), jnp.float32),
        compiler_params=pltpu.CompilerParams(
            dimension_semantics=("parallel",),
        ),
    )(bc3, t3)
    return jnp.sum(per_batch[:, 0, 0]) / n


# speedup vs baseline: 1.4085x; 1.4085x over previous
import jax
import jax.numpy as jnp
from jax.experimental import pallas as pl
from jax.experimental.pallas import tpu as pltpu

_P = 256
_QB = 1920
_BIG = 1e9


def _chamfer_body(bc_ref, t_ref, out_ref):
    bc = bc_ref[0]
    q = t_ref.shape[2]
    nchunks = q // _QB

    def body(j, carry):
        run_min, acc_y, acc_len = carry
        tj = t_ref[0, :, pl.ds(j * _QB, _QB)]
        mask = tj >= 0.001
        tx = jnp.where(mask, tj, _BIG)
        d = (bc - tx) ** 2
        run_min = jnp.minimum(run_min, jnp.min(d, axis=1, keepdims=True))
        dy = jnp.min(d, axis=0, keepdims=True)
        acc_y = acc_y + jnp.where(mask, dy, 0.0)
        acc_len = acc_len + mask.astype(jnp.float32)
        return run_min, acc_y, acc_len

    init = (
        jnp.full((_P, 1), jnp.inf, jnp.float32),
        jnp.zeros((1, _QB), jnp.float32),
        jnp.zeros((1, _QB), jnp.float32),
    )
    run_min, acc_y, acc_len = jax.lax.fori_loop(0, nchunks, body, init)
    cham_x = jnp.sum(run_min) / _P
    cham_y = jnp.sum(acc_y) / jnp.maximum(jnp.sum(acc_len), 1.0)
    out_ref[0] = jnp.full((1, 128), cham_x + cham_y, jnp.float32)


def kernel(bins, target_depth_maps):
    n = bins.shape[0]
    q = target_depth_maps.shape[1] * target_depth_maps.shape[2]
    bc = 0.5 * (bins[:, 1:] + bins[:, :-1])
    bc3 = bc.reshape(n, _P, 1)
    t3 = target_depth_maps.reshape(n, 1, q)
    per_batch = pl.pallas_call(
        _chamfer_body,
        grid=(n,),
        in_specs=[
            pl.BlockSpec((1, _P, 1), lambda i: (i, 0, 0)),
            pl.BlockSpec((1, 1, q), lambda i: (i, 0, 0)),
        ],
        out_specs=pl.BlockSpec((1, 1, 128), lambda i: (i, 0, 0)),
        out_shape=jax.ShapeDtypeStruct((n, 1, 128), jnp.float32),
        compiler_params=pltpu.CompilerParams(
            dimension_semantics=("parallel",),
        ),
    )(bc3, t3)
    return jnp.sum(per_batch[:, 0, 0]) / n
